# Initial kernel scaffold; baseline (speedup 1.0000x reference)
import jax
import jax.numpy as jnp
from jax.experimental import pallas as pl


def kernel(v, c, v_sem, c_sem, v_class, c_class, Ws_v, bs_v, Ws_c, bs_c, Wq_v, bq_v, Wk_v, bk_v, Wv_v, bv_v, Wo_v, bo_v, Wq_c, bq_c, Wk_c, bk_c, Wv_c, bv_c, Wo_c, bo_c, Wrec_v, brec_v, Wrec_c, brec_c, Wgate_v, bgate_v, Wgate_c, bgate_c, gamma, beta):
    raise NotImplementedError("write your pallas kernel here")


# speedup vs baseline: 2.8860x; 2.8860x over previous
import functools

import jax
import jax.numpy as jnp
from jax.experimental import pallas as pl
from jax.experimental.pallas import tpu as pltpu

NCLS = 32
EMBD = 64
HD = 4
DHD = 16
BLK = 2000


def _dg(a, b, ca, cb):
    return jax.lax.dot_general(
        a, b, (((ca,), (cb,)), ((), ())), preferred_element_type=jnp.float32)


def _pass1_body(x_ref, cls_ref, sem_ref, Ws_ref, bs_ref, Wq_ref, bq_ref,
                Wk_ref, bk_ref, Wv_ref, bv_ref, Wo_ref, bo_ref, WrecA_ref,
                WrecB_ref, brec_ref, WgA_ref, WgB_ref, bg_ref, gamma_ref,
                beta_ref, fused_ref, q_s, mT_s, dT_s, oT_s, sxsT_s, cnt_s,
                *, nblk, blk):
    i = pl.program_id(0)

    @pl.when(i == 0)
    def _init():
        q_s[...] = _dg(sem_ref[...], Wq_ref[...], 1, 1) + bq_ref[...]
        mT_s[...] = jnp.full((HD, NCLS), -3e38, jnp.float32)
        dT_s[...] = jnp.zeros((HD, NCLS), jnp.float32)
        oT_s[...] = jnp.zeros((EMBD, NCLS), jnp.float32)
        sxsT_s[...] = jnp.zeros((EMBD, NCLS), jnp.float32)
        cnt_s[...] = jnp.zeros((1, NCLS), jnp.float32)

    xb = x_ref[...]
    cls2 = cls_ref[0]
    onehot = (jax.lax.broadcasted_iota(jnp.int32, (NCLS, blk), 0)
              == cls2).astype(jnp.float32)

    xs = _dg(xb, Ws_ref[...], 1, 1) + bs_ref[...]
    k = _dg(xs, Wk_ref[...], 1, 1) + bk_ref[...]
    vv = _dg(xs, Wv_ref[...], 1, 1) + bv_ref[...]

    qg = _dg(onehot, q_s[...], 0, 0)
    prod = k * qg
    esel = (jax.lax.broadcasted_iota(jnp.int32, (EMBD, HD), 0) // DHD
            == jax.lax.broadcasted_iota(jnp.int32, (EMBD, HD), 1)
            ).astype(jnp.float32)
    sT = _dg(esel, prod, 0, 1) * (1.0 / 4.0)

    masked = jnp.where(onehot[None, :, :] > 0.0, sT[:, None, :], -3e38)
    mblkT = jnp.max(masked, axis=2)
    mT = mT_s[...]
    mnewT = jnp.maximum(mT, mblkT)
    scaleT = jnp.exp(mT - mnewT)
    mT_s[...] = mnewT
    mgT = _dg(mnewT, onehot, 1, 0)
    wT = jnp.exp(sT - mgT)
    dT_s[...] = dT_s[...] * scaleT + _dg(wT, onehot, 1, 1)
    for h in range(HD):
        wh = onehot * wT[h:h + 1, :]
        oT_s[h * DHD:(h + 1) * DHD, :] = (
            oT_s[h * DHD:(h + 1) * DHD, :] * scaleT[h:h + 1, :]
            + _dg(vv[:, h * DHD:(h + 1) * DHD], wh, 0, 1))
    sxsT_s[...] = sxsT_s[...] + _dg(xs, onehot, 0, 1)
    cnt_s[...] = cnt_s[...] + _dg(jnp.ones((1, blk), jnp.float32), onehot, 1, 1)

    @pl.when(i == nblk - 1)
    def _epilogue():
        attT = oT_s[...] / jnp.maximum(
            jnp.repeat(dT_s[...], DHD, axis=0), 1e-30)
        attout = _dg(attT, Wo_ref[...], 0, 1) + bo_ref[...]
        new_fea = (_dg(sem_ref[...], WrecA_ref[...], 1, 1)
                   + _dg(attout, WrecB_ref[...], 1, 1) + brec_ref[...])
        oldT = sxsT_s[...] / jnp.maximum(cnt_s[...], 1.0)
        eye = (jax.lax.broadcasted_iota(jnp.int32, (EMBD, EMBD), 0)
               == jax.lax.broadcasted_iota(jnp.int32, (EMBD, EMBD), 1)
               ).astype(jnp.float32)
        old_fea = _dg(oldT, eye, 0, 1)
        glogit = (_dg(oldT, WgA_ref[...], 0, 1)
                  + _dg(new_fea, WgB_ref[...], 1, 1) + bg_ref[...])
        gate = 1.0 / (1.0 + jnp.exp(-glogit))
        fused = gate * old_fea + (1.0 - gate) * new_fea
        mu = jnp.mean(fused, axis=-1, keepdims=True)
        var = jnp.mean((fused - mu) ** 2, axis=-1, keepdims=True)
        fused_ref[...] = ((fused - mu) * jax.lax.rsqrt(var + 1e-5)
                          * gamma_ref[...] + beta_ref[...])


def _pass2_body(x_ref, cls_ref, fused_ref, out_ref, *, blk):
    cls2 = cls_ref[0]
    onehot = (jax.lax.broadcasted_iota(jnp.int32, (NCLS, blk), 0)
              == cls2).astype(jnp.float32)
    out_ref[...] = _dg(onehot, fused_ref[...], 0, 0) * x_ref[...]


def _full(shape):
    return pl.BlockSpec(shape, lambda i: tuple(0 for _ in shape))


def _branch(x, sem, cls, Ws, bs, Wq, bq, Wk, bk, Wv, bv, Wo, bo,
            Wrec, brec, Wgate, bgate, gamma, beta):
    n = x.shape[0]
    blk = BLK
    nblk = n // blk
    assert nblk * blk == n
    cls3 = cls.astype(jnp.int32).reshape(nblk, 1, blk)
    r = lambda a: a.reshape(1, EMBD)
    WrecA, WrecB = Wrec[:, :EMBD], Wrec[:, EMBD:]
    WgA, WgB = Wgate[:, :EMBD], Wgate[:, EMBD:]

    w64 = _full((EMBD, EMBD))
    b1 = _full((1, EMBD))
    fused = pl.pallas_call(
        functools.partial(_pass1_body, nblk=nblk, blk=blk),
        grid=(nblk,),
        in_specs=[
            pl.BlockSpec((blk, EMBD), lambda i: (i, 0)),
            pl.BlockSpec((1, 1, blk), lambda i: (i, 0, 0)),
            _full((NCLS, EMBD)), w64, b1, w64, b1, w64, b1, w64, b1,
            w64, b1, w64, w64, b1, w64, w64, b1, b1, b1,
        ],
        out_specs=_full((NCLS, EMBD)),
        out_shape=jax.ShapeDtypeStruct((NCLS, EMBD), jnp.float32),
        scratch_shapes=[
            pltpu.VMEM((NCLS, EMBD), jnp.float32),
            pltpu.VMEM((HD, NCLS), jnp.float32),
            pltpu.VMEM((HD, NCLS), jnp.float32),
            pltpu.VMEM((EMBD, NCLS), jnp.float32),
            pltpu.VMEM((EMBD, NCLS), jnp.float32),
            pltpu.VMEM((1, NCLS), jnp.float32),
        ],
    )(x, cls3, sem, Ws, r(bs), Wq, r(bq), Wk, r(bk), Wv, r(bv), Wo, r(bo),
      WrecA, WrecB, r(brec), WgA, WgB, r(bgate), r(gamma), r(beta))

    upd = pl.pallas_call(
        functools.partial(_pass2_body, blk=blk),
        grid=(nblk,),
        in_specs=[
            pl.BlockSpec((blk, EMBD), lambda i: (i, 0)),
            pl.BlockSpec((1, 1, blk), lambda i: (i, 0, 0)),
            _full((NCLS, EMBD)),
        ],
        out_specs=pl.BlockSpec((blk, EMBD), lambda i: (i, 0)),
        out_shape=jax.ShapeDtypeStruct((n, EMBD), jnp.float32),
    )(x, cls3, fused)
    return upd


def kernel(v, c, v_sem, c_sem, v_class, c_class, Ws_v, bs_v, Ws_c, bs_c,
           Wq_v, bq_v, Wk_v, bk_v, Wv_v, bv_v, Wo_v, bo_v, Wq_c, bq_c,
           Wk_c, bk_c, Wv_c, bv_c, Wo_c, bo_c, Wrec_v, brec_v, Wrec_c,
           brec_c, Wgate_v, bgate_v, Wgate_c, bgate_c, gamma, beta):
    v_upd = _branch(v, v_sem, v_class, Ws_v, bs_v, Wq_v, bq_v, Wk_v, bk_v,
                    Wv_v, bv_v, Wo_v, bo_v, Wrec_v, brec_v, Wgate_v, bgate_v,
                    gamma, beta)
    c_upd = _branch(c, c_sem, c_class, Ws_c, bs_c, Wq_c, bq_c, Wk_c, bk_c,
                    Wv_c, bv_c, Wo_c, bo_c, Wrec_c, brec_c, Wgate_c, bgate_c,
                    gamma, beta)
    return (v_upd, c_upd)
